# baseline (speedup 1.0000x reference)
import functools

import jax
import jax.numpy as jnp
import numpy as np
from jax.experimental import pallas as pl
from jax.experimental.pallas import tpu as pltpu

_B, _C, _N = 64, 32, 4096
_HALF = _B // 2


@functools.lru_cache(maxsize=1)
def _gumbel_const_half():
    eps = 1e-20
    u = jax.random.uniform(jax.random.key(42), (_B, _C, _N), dtype=jnp.float32)
    g = -jnp.log(-jnp.log(u + eps) + eps)
    return jnp.array(g[:_HALF])


def _threefry_gumbel(batch):
    base = (batch * (_C * _N)).astype(jnp.uint32)
    row = jax.lax.broadcasted_iota(jnp.uint32, (_C, _N), 0) * jnp.uint32(_N)
    col = jax.lax.broadcasted_iota(jnp.uint32, (_C, _N), 1)
    i = base + row + col

    ks0 = jnp.uint32(0)
    ks1 = jnp.uint32(42)
    ks2 = jnp.uint32(0x1BD11BDA) ^ ks0 ^ ks1
    ks = (ks0, ks1, ks2)
    rots = ((13, 15, 26, 6), (17, 29, 16, 24))
    x0 = jnp.zeros((_C, _N), jnp.uint32) + ks0
    x1 = i + ks1
    for grp in range(5):
        for r in rots[grp % 2]:
            x0 = x0 + x1
            x1 = (x1 << jnp.uint32(r)) | (x1 >> jnp.uint32(32 - r))
            x1 = x0 ^ x1
        x0 = x0 + ks[(grp + 1) % 3]
        x1 = x1 + ks[(grp + 2) % 3] + jnp.uint32(grp + 1)
    bits = x0 ^ x1

    fb = (bits >> jnp.uint32(9)) | jnp.uint32(0x3F800000)
    u = jax.lax.bitcast_convert_type(fb, jnp.float32) - jnp.float32(1.0)
    eps = jnp.float32(1e-20)
    return -jnp.log(-jnp.log(u + eps) + eps)


def _body(l_ref, g_ref, o_ref):
    b = pl.program_id(0)
    l = l_ref[0]
    m = jnp.max(l, axis=0, keepdims=True)
    e = jnp.exp(l - m)
    p = e / jnp.sum(e, axis=0, keepdims=True)
    logp = jnp.log(p)

    @pl.when(b % 2 == 0)
    def _const_branch():
        z = logp + g_ref[0]
        o_ref[0] = (z[0:1, :] >= jnp.max(z, axis=0, keepdims=True)).astype(
            jnp.float32)

    @pl.when(b % 2 == 1)
    def _threefry_branch():
        g = _threefry_gumbel(_HALF + b // 2)
        z = logp + g
        o_ref[0] = (z[0:1, :] >= jnp.max(z, axis=0, keepdims=True)).astype(
            jnp.float32)


def kernel(logits):
    gc = _gumbel_const_half()
    grid = (_B,)
    out = pl.pallas_call(
        _body,
        grid=grid,
        in_specs=[
            pl.BlockSpec((1, _C, _N), lambda b: (b // 2 + _HALF * (b % 2), 0, 0)),
            pl.BlockSpec((1, _C, _N), lambda b: (b // 2, 0, 0)),
        ],
        out_specs=pl.BlockSpec((1, 1, _N), lambda b: (b // 2 + _HALF * (b % 2), 0, 0)),
        out_shape=jax.ShapeDtypeStruct((_B, 1, _N), jnp.float32),
        compiler_params=pltpu.CompilerParams(
            dimension_semantics=("arbitrary",),
        ),
    )(logits, gc)
    return out


# speedup vs baseline: 1.4678x; 1.4678x over previous
import jax
import jax.numpy as jnp
from jax.experimental import pallas as pl
from jax.experimental.pallas import tpu as pltpu

_B, _C, _N = 64, 32, 4096


def _threefry_gumbel(batch):
    base = (batch * (_C * _N)).astype(jnp.uint32)
    row = jax.lax.broadcasted_iota(jnp.uint32, (_C, _N), 0) * jnp.uint32(_N)
    col = jax.lax.broadcasted_iota(jnp.uint32, (_C, _N), 1)
    i = base + row + col

    ks0 = jnp.uint32(0)
    ks1 = jnp.uint32(42)
    ks2 = jnp.uint32(0x1BD11BDA) ^ ks0 ^ ks1
    ks = (ks0, ks1, ks2)
    rots = ((13, 15, 26, 6), (17, 29, 16, 24))
    x0 = jnp.zeros((_C, _N), jnp.uint32) + ks0
    x1 = i + ks1
    for grp in range(5):
        for r in rots[grp % 2]:
            x0 = x0 + x1
            x1 = (x1 << jnp.uint32(r)) | (x1 >> jnp.uint32(32 - r))
            x1 = x0 ^ x1
        x0 = x0 + ks[(grp + 1) % 3]
        x1 = x1 + ks[(grp + 2) % 3] + jnp.uint32(grp + 1)
    bits = x0 ^ x1

    fb = (bits >> jnp.uint32(9)) | jnp.uint32(0x3F800000)
    u = jax.lax.bitcast_convert_type(fb, jnp.float32) - jnp.float32(1.0)
    eps = jnp.float32(1e-20)
    return -jnp.log(-jnp.log(u + eps) + eps)


def _body(l_ref, o_ref):
    b = pl.program_id(0)
    l = l_ref[0]
    m = jnp.max(l, axis=0, keepdims=True)
    e = jnp.exp(l - m)
    p = e / jnp.sum(e, axis=0, keepdims=True)
    logp = jnp.log(p)
    z = logp + _threefry_gumbel(b)
    o_ref[0] = (z[0:1, :] >= jnp.max(z, axis=0, keepdims=True)).astype(
        jnp.float32)


def kernel(logits):
    return pl.pallas_call(
        _body,
        grid=(_B,),
        in_specs=[pl.BlockSpec((1, _C, _N), lambda b: (b, 0, 0))],
        out_specs=pl.BlockSpec((1, 1, _N), lambda b: (b, 0, 0)),
        out_shape=jax.ShapeDtypeStruct((_B, 1, _N), jnp.float32),
        compiler_params=pltpu.CompilerParams(
            dimension_semantics=("arbitrary",),
        ),
    )(logits)
